# baseline (speedup 1.0000x reference)
import functools

import jax
import jax.numpy as jnp
from jax.experimental import pallas as pl
from jax.experimental.pallas import tpu as pltpu

BLK = 8192


def _body(n_cols, x_ref, w_ref, b_ref, o_ref, acc_ref):
    pid = pl.program_id(0)

    @pl.when(pid == 0)
    def _init():
        acc_ref[...] = jnp.zeros_like(acc_ref)

    x = x_ref[...]
    w = w_ref[...]
    col = pl.program_id(0) * BLK + jax.lax.broadcasted_iota(
        jnp.int32, x.shape, 1)
    prod = jnp.where(col < n_cols, x * w, 0.0)
    rows = x.shape[0]
    acc_ref[...] += prod.reshape(rows, BLK // 128, 128).sum(axis=1)

    @pl.when(pid == pl.num_programs(0) - 1)
    def _fini():
        o_ref[...] = acc_ref[...].sum(axis=1, keepdims=True) + b_ref[0]


def kernel(inputs, W, b):
    rows, n = inputs.shape
    grid = pl.cdiv(n, BLK)
    w_row = W.reshape(1, n)
    b_arr = jnp.asarray(b, jnp.float32).reshape(1)
    out = pl.pallas_call(
        functools.partial(_body, n),
        grid=(grid,),
        in_specs=[
            pl.BlockSpec((rows, BLK), lambda i: (0, i)),
            pl.BlockSpec((1, BLK), lambda i: (0, i)),
            pl.BlockSpec(memory_space=pltpu.SMEM),
        ],
        out_specs=pl.BlockSpec((rows, 1), lambda i: (0, 0)),
        out_shape=jax.ShapeDtypeStruct((rows, 1), jnp.float32),
        scratch_shapes=[pltpu.VMEM((rows, 128), jnp.float32)],
        compiler_params=pltpu.CompilerParams(
            dimension_semantics=("arbitrary",)),
    )(inputs, w_row, b_arr)
    return out


# speedup vs baseline: 1.2293x; 1.2293x over previous
import functools

import jax
import jax.numpy as jnp
from jax.experimental import pallas as pl
from jax.experimental.pallas import tpu as pltpu

BLK = 8192


def _body(n_cols, x_ref, w_ref, b_ref, o_ref, acc_ref):
    pid = pl.program_id(0)
    last = pl.num_programs(0) - 1

    @pl.when(pid == 0)
    def _init():
        acc_ref[...] = jnp.zeros_like(acc_ref)

    @pl.when(pid != last)
    def _main():
        acc_ref[...] += x_ref[...] * w_ref[...]

    @pl.when(pid == last)
    def _fini():
        x = x_ref[...]
        w = w_ref[...]
        col = pid * BLK + jax.lax.broadcasted_iota(jnp.int32, x.shape, 1)
        acc_ref[...] += jnp.where(col < n_cols, x * w, 0.0)
        o_ref[...] = acc_ref[...].sum(axis=1, keepdims=True) + b_ref[0]


def kernel(inputs, W, b):
    rows, n = inputs.shape
    grid = pl.cdiv(n, BLK)
    w_row = W.reshape(1, n)
    b_arr = jnp.asarray(b, jnp.float32).reshape(1)
    out = pl.pallas_call(
        functools.partial(_body, n),
        grid=(grid,),
        in_specs=[
            pl.BlockSpec((rows, BLK), lambda i: (0, i)),
            pl.BlockSpec((1, BLK), lambda i: (0, i)),
            pl.BlockSpec(memory_space=pltpu.SMEM),
        ],
        out_specs=pl.BlockSpec((rows, 1), lambda i: (0, 0)),
        out_shape=jax.ShapeDtypeStruct((rows, 1), jnp.float32),
        scratch_shapes=[pltpu.VMEM((rows, BLK), jnp.float32)],
        compiler_params=pltpu.CompilerParams(
            dimension_semantics=("arbitrary",)),
    )(inputs, w_row, b_arr)
    return out


# speedup vs baseline: 1.2417x; 1.0101x over previous
import functools

import jax
import jax.numpy as jnp
from jax.experimental import pallas as pl
from jax.experimental.pallas import tpu as pltpu

BLK = 8192


def _body(n_cols, x_ref, w_ref, b_ref, o_ref, acc_ref):
    pid = pl.program_id(0)
    last = pl.num_programs(0) - 1

    @pl.when(pid == 0)
    def _init():
        acc_ref[...] = jnp.zeros_like(acc_ref)

    def _lane_group_sum(prod):
        parts = [prod[:, g * 128:(g + 1) * 128] for g in range(BLK // 128)]
        while len(parts) > 1:
            parts = [a + b for a, b in zip(parts[0::2], parts[1::2])] + (
                [parts[-1]] if len(parts) % 2 else [])
        return parts[0]

    @pl.when(pid != last)
    def _main():
        acc_ref[...] += _lane_group_sum(x_ref[...] * w_ref[...])

    @pl.when(pid == last)
    def _fini():
        x = x_ref[...]
        w = w_ref[...]
        col = pid * BLK + jax.lax.broadcasted_iota(jnp.int32, x.shape, 1)
        acc_ref[...] += _lane_group_sum(jnp.where(col < n_cols, x * w, 0.0))
        o_ref[...] = acc_ref[...].sum(axis=1, keepdims=True) + b_ref[0]


def kernel(inputs, W, b):
    rows, n = inputs.shape
    grid = pl.cdiv(n, BLK)
    w_row = W.reshape(1, n)
    b_arr = jnp.asarray(b, jnp.float32).reshape(1)
    out = pl.pallas_call(
        functools.partial(_body, n),
        grid=(grid,),
        in_specs=[
            pl.BlockSpec((rows, BLK), lambda i: (0, i)),
            pl.BlockSpec((1, BLK), lambda i: (0, i)),
            pl.BlockSpec(memory_space=pltpu.SMEM),
        ],
        out_specs=pl.BlockSpec((rows, 1), lambda i: (0, 0)),
        out_shape=jax.ShapeDtypeStruct((rows, 1), jnp.float32),
        scratch_shapes=[pltpu.VMEM((rows, 128), jnp.float32)],
        compiler_params=pltpu.CompilerParams(
            dimension_semantics=("arbitrary",)),
    )(inputs, w_row, b_arr)
    return out


# speedup vs baseline: 1.8721x; 1.5077x over previous
import functools

import jax
import jax.numpy as jnp
from jax.experimental import pallas as pl
from jax.experimental.pallas import tpu as pltpu

BLK = 32768


def _body(n_cols, x_ref, w_ref, b_ref, o_ref, acc_ref):
    pid = pl.program_id(0)
    last = pl.num_programs(0) - 1

    @pl.when(pid == 0)
    def _init():
        acc_ref[...] = jnp.zeros_like(acc_ref)

    def _lane_group_sum(prod):
        parts = [prod[:, g * 128:(g + 1) * 128] for g in range(BLK // 128)]
        while len(parts) > 1:
            parts = [a + b for a, b in zip(parts[0::2], parts[1::2])] + (
                [parts[-1]] if len(parts) % 2 else [])
        return parts[0]

    @pl.when(pid != last)
    def _main():
        acc_ref[...] += _lane_group_sum(x_ref[...] * w_ref[...])

    @pl.when(pid == last)
    def _fini():
        x = x_ref[...]
        w = w_ref[...]
        col = pid * BLK + jax.lax.broadcasted_iota(jnp.int32, x.shape, 1)
        acc_ref[...] += _lane_group_sum(jnp.where(col < n_cols, x * w, 0.0))
        o_ref[...] = acc_ref[...].sum(axis=1, keepdims=True) + b_ref[0]


def kernel(inputs, W, b):
    rows, n = inputs.shape
    grid = pl.cdiv(n, BLK)
    w_row = W.reshape(1, n)
    b_arr = jnp.asarray(b, jnp.float32).reshape(1)
    out = pl.pallas_call(
        functools.partial(_body, n),
        grid=(grid,),
        in_specs=[
            pl.BlockSpec((rows, BLK), lambda i: (0, i)),
            pl.BlockSpec((1, BLK), lambda i: (0, i)),
            pl.BlockSpec(memory_space=pltpu.SMEM),
        ],
        out_specs=pl.BlockSpec((rows, 1), lambda i: (0, 0)),
        out_shape=jax.ShapeDtypeStruct((rows, 1), jnp.float32),
        scratch_shapes=[pltpu.VMEM((rows, 128), jnp.float32)],
        compiler_params=pltpu.CompilerParams(
            dimension_semantics=("arbitrary",)),
    )(inputs, w_row, b_arr)
    return out


# speedup vs baseline: 1.8758x; 1.0020x over previous
import functools

import jax
import jax.numpy as jnp
from jax.experimental import pallas as pl
from jax.experimental.pallas import tpu as pltpu

BLK = 65536


def _body(n_cols, x_ref, w_ref, b_ref, o_ref, acc_ref):
    pid = pl.program_id(0)
    last = pl.num_programs(0) - 1

    @pl.when(pid == 0)
    def _init():
        acc_ref[...] = jnp.zeros_like(acc_ref)

    def _lane_group_sum(prod):
        parts = [prod[:, g * 128:(g + 1) * 128] for g in range(BLK // 128)]
        while len(parts) > 1:
            parts = [a + b for a, b in zip(parts[0::2], parts[1::2])] + (
                [parts[-1]] if len(parts) % 2 else [])
        return parts[0]

    @pl.when(pid != last)
    def _main():
        acc_ref[...] += _lane_group_sum(x_ref[...] * w_ref[...])

    @pl.when(pid == last)
    def _fini():
        x = x_ref[...]
        w = w_ref[...]
        col = pid * BLK + jax.lax.broadcasted_iota(jnp.int32, x.shape, 1)
        acc_ref[...] += _lane_group_sum(jnp.where(col < n_cols, x * w, 0.0))
        o_ref[...] = acc_ref[...].sum(axis=1, keepdims=True) + b_ref[0]


def kernel(inputs, W, b):
    rows, n = inputs.shape
    grid = pl.cdiv(n, BLK)
    w_row = W.reshape(1, n)
    b_arr = jnp.asarray(b, jnp.float32).reshape(1)
    out = pl.pallas_call(
        functools.partial(_body, n),
        grid=(grid,),
        in_specs=[
            pl.BlockSpec((rows, BLK), lambda i: (0, i)),
            pl.BlockSpec((1, BLK), lambda i: (0, i)),
            pl.BlockSpec(memory_space=pltpu.SMEM),
        ],
        out_specs=pl.BlockSpec((rows, 1), lambda i: (0, 0)),
        out_shape=jax.ShapeDtypeStruct((rows, 1), jnp.float32),
        scratch_shapes=[pltpu.VMEM((rows, 128), jnp.float32)],
        compiler_params=pltpu.CompilerParams(
            dimension_semantics=("arbitrary",)),
    )(inputs, w_row, b_arr)
    return out
